# Initial kernel scaffold; baseline (speedup 1.0000x reference)
import jax
import jax.numpy as jnp
from jax.experimental import pallas as pl


def kernel(object_X, event_X, W_ev, b_ev, g_ev, be_ev, W_ob, b_ob, g_ob, be_ob, W1, b1, W2, b2, Wg, bg, node_idx, hedge_idx, main_object, event_sel):
    raise NotImplementedError("write your pallas kernel here")


# speedup vs baseline: 6.4398x; 6.4398x over previous
import functools

import jax
import jax.numpy as jnp
from jax import lax
from jax.experimental import pallas as pl
from jax.experimental.pallas import tpu as pltpu
from jax.experimental.pallas import tpu_sc as plsc

N_EVENTS = 6000
N_OBJECTS = 4000
N_NODES = 10000
N_HEDGES = 2000
N_INC = 320000
D = 128

NC = 2
NS = 16
NW = NC * NS
PER_TILE = N_INC // NW
K = 80
CH = PER_TILE // K

NPAD = 10240
HPAD = 2048

_mesh = lambda: plsc.VectorSubcoreMesh(
    core_axis_name="c", subcore_axis_name="s", num_cores=NC, num_subcores=NS)

_sc_params = lambda: pltpu.CompilerParams(needs_layout_passes=False)


def _counts_body(gn, gh, outn, outh, gn_v, gh_v, ncnt_v, ecnt_v):
  c = lax.axis_index("c")
  s = lax.axis_index("s")
  wid = c * NS + s
  pltpu.sync_copy(gn.at[wid], gn_v)
  pltpu.sync_copy(gh.at[wid], gh_v)
  zeros16 = jnp.zeros((16,), jnp.float32)

  def zn(i, _):
    ncnt_v[pl.ds(i * 16, 16)] = zeros16
    return 0
  lax.fori_loop(0, NPAD // 16, zn, 0)

  def zh(i, _):
    ecnt_v[pl.ds(i * 16, 16)] = zeros16
    return 0
  lax.fori_loop(0, HPAD // 16, zh, 0)

  ones16 = jnp.ones((16,), jnp.float32)

  def crow(j, _):
    for g in range(K // 16):
      vn = gn_v[j, pl.ds(g * 16, 16)]
      plsc.addupdate_scatter(ncnt_v, [vn], ones16)
      vh = gh_v[j, pl.ds(g * 16, 16)]
      plsc.addupdate_scatter(ecnt_v, [vh], ones16)
    return 0
  lax.fori_loop(0, CH, crow, 0)

  pltpu.sync_copy(ncnt_v, outn.at[wid])
  pltpu.sync_copy(ecnt_v, outh.at[wid])


@jax.jit
def _sc_counts(gn, gh):
  return pl.kernel(
      _counts_body,
      out_type=(jax.ShapeDtypeStruct((NW, NPAD), jnp.float32),
                jax.ShapeDtypeStruct((NW, HPAD), jnp.float32)),
      mesh=_mesh(),
      compiler_params=_sc_params(),
      scratch_types=[
          pltpu.VMEM((CH, K), jnp.int32),
          pltpu.VMEM((CH, K), jnp.int32),
          pltpu.VMEM((NPAD,), jnp.float32),
          pltpu.VMEM((HPAD,), jnp.float32),
      ],
  )(gn, gh)


def _seg_body(opad, table, gidx, sidx, zbuf, out, gidx_v, sidx_v, rows_v, accum):
  c = lax.axis_index("c")
  s = lax.axis_index("s")
  wid = c * NS + s
  pltpu.sync_copy(gidx.at[wid], gidx_v)
  pltpu.sync_copy(sidx.at[wid], sidx_v)
  zr = opad // NS
  pltpu.sync_copy(zbuf, accum.at[pl.ds(s * zr, zr)])
  plsc.subcore_barrier()

  def chunk(j, _):
    pltpu.sync_copy(table.at[gidx_v.at[j]], rows_v)
    pltpu.sync_copy(rows_v, accum.at[sidx_v.at[j]], add=True)
    return 0
  lax.fori_loop(0, CH, chunk, 0)

  plsc.subcore_barrier()
  pltpu.sync_copy(accum.at[pl.ds(s * zr, zr)], out.at[c, pl.ds(s * zr, zr)])


@functools.partial(jax.jit, static_argnames=("opad",))
def _sc_seg(table, gidx, sidx, zbuf, opad):
  return pl.kernel(
      functools.partial(_seg_body, opad),
      out_type=jax.ShapeDtypeStruct((NC, opad, D), jnp.float32),
      mesh=_mesh(),
      compiler_params=_sc_params(),
      scratch_types=[
          pltpu.VMEM((CH, K), jnp.int32),
          pltpu.VMEM((CH, K), jnp.int32),
          pltpu.VMEM((K, D), jnp.float32),
          pltpu.VMEM_SHARED((opad, D), jnp.float32),
      ],
  )(table, gidx, sidx, zbuf)


def _gather_body(table, idx, out, idx_v, rows_v, sem):
  c = lax.axis_index("c")
  s = lax.axis_index("s")
  wid = c * NS + s
  bpw = 2048 // NW
  base = wid * bpw
  pltpu.sync_copy(idx.at[pl.ds(base, bpw)], idx_v)
  pltpu.async_copy(table.at[idx_v], rows_v, sem).wait()
  pltpu.sync_copy(rows_v, out.at[pl.ds(base, bpw)])


@jax.jit
def _sc_gather(table, idx):
  bpw = 2048 // NW
  return pl.kernel(
      _gather_body,
      out_type=jax.ShapeDtypeStruct((2048, D), jnp.float32),
      mesh=_mesh(),
      compiler_params=_sc_params(),
      scratch_types=[
          pltpu.VMEM((bpw,), jnp.int32),
          pltpu.VMEM((bpw, D), jnp.float32),
          pltpu.SemaphoreType.DMA,
      ],
  )(table, idx)


def _proj_body(x_ref, w_ref, b_ref, g_ref, be_ref, o_ref):
  y = jnp.dot(x_ref[...], w_ref[...], preferred_element_type=jnp.float32)
  y = y + b_ref[...]
  y = jnp.where(y >= 0, y, 0.2 * y)
  m = jnp.mean(y, axis=-1, keepdims=True)
  v = jnp.mean((y - m) ** 2, axis=-1, keepdims=True)
  o_ref[...] = (y - m) / jnp.sqrt(v + 1e-5) * g_ref[...] + be_ref[...]


@jax.jit
def _proj(x, w, b, g, be):
  n = x.shape[0]
  rb = 1000
  grid = n // rb
  return pl.pallas_call(
      _proj_body,
      grid=(grid,),
      in_specs=[
          pl.BlockSpec((rb, D), lambda i: (i, 0)),
          pl.BlockSpec((D, D), lambda i: (0, 0)),
          pl.BlockSpec((1, D), lambda i: (0, 0)),
          pl.BlockSpec((1, D), lambda i: (0, 0)),
          pl.BlockSpec((1, D), lambda i: (0, 0)),
      ],
      out_specs=pl.BlockSpec((rb, D), lambda i: (i, 0)),
      out_shape=jax.ShapeDtypeStruct((n, D), jnp.float32),
  )(x, w, b.reshape(1, D), g.reshape(1, D), be.reshape(1, D))


def _combine_body(p_ref, c_ref, o_ref):
  cnt = jnp.maximum(jnp.sum(c_ref[...], axis=0), 1.0)
  o_ref[...] = (p_ref[0] + p_ref[1]) * (1.0 / cnt)[:, None]


@jax.jit
def _combine(parts, cparts):
  rb = 256
  grid = HPAD // rb
  return pl.pallas_call(
      _combine_body,
      grid=(grid,),
      in_specs=[
          pl.BlockSpec((NC, rb, D), lambda i: (0, i, 0)),
          pl.BlockSpec((NW, rb), lambda i: (0, i)),
      ],
      out_specs=pl.BlockSpec((rb, D), lambda i: (i, 0)),
      out_shape=jax.ShapeDtypeStruct((HPAD, D), jnp.float32),
  )(parts, cparts)


def _update_body(p_ref, c_ref, x_ref, w_ref, b_ref, o_ref):
  cnt = jnp.maximum(jnp.sum(c_ref[...], axis=0), 1.0)
  nf = (p_ref[0] + p_ref[1]) * (1.0 / cnt)[:, None]
  y = jnp.dot(nf, w_ref[...], preferred_element_type=jnp.float32) + b_ref[...]
  o_ref[...] = jnp.maximum(y, 0.0) + x_ref[...]


@jax.jit
def _update(parts, cparts, xres, w, b):
  rb = 1024
  grid = NPAD // rb
  return pl.pallas_call(
      _update_body,
      grid=(grid,),
      in_specs=[
          pl.BlockSpec((NC, rb, D), lambda i: (0, i, 0)),
          pl.BlockSpec((NW, rb), lambda i: (0, i)),
          pl.BlockSpec((rb, D), lambda i: (i, 0)),
          pl.BlockSpec((D, D), lambda i: (0, 0)),
          pl.BlockSpec((1, D), lambda i: (0, 0)),
      ],
      out_specs=pl.BlockSpec((rb, D), lambda i: (i, 0)),
      out_shape=jax.ShapeDtypeStruct((N_NODES, D), jnp.float32),
  )(parts, cparts, xres, w, b.reshape(1, D))


def _fusion_body(ev_ref, ob_ref, w1_ref, w2_ref, b_ref, o_ref):
  ev = ev_ref[...]
  ob = ob_ref[...]
  z = (jnp.dot(ob, w1_ref[...], preferred_element_type=jnp.float32)
       + jnp.dot(ev, w2_ref[...], preferred_element_type=jnp.float32)
       + b_ref[...])
  g = jax.nn.sigmoid(z)
  o_ref[...] = g * ob + (1.0 - g) * ev


@jax.jit
def _fusion(ev, ob, w1, w2, b):
  n = ev.shape[0]
  return pl.pallas_call(
      _fusion_body,
      grid=(1,),
      in_specs=[
          pl.BlockSpec((n, D), lambda i: (0, 0)),
          pl.BlockSpec((n, D), lambda i: (0, 0)),
          pl.BlockSpec((D, D), lambda i: (0, 0)),
          pl.BlockSpec((D, D), lambda i: (0, 0)),
          pl.BlockSpec((1, D), lambda i: (0, 0)),
      ],
      out_specs=pl.BlockSpec((n, D), lambda i: (0, 0)),
      out_shape=jax.ShapeDtypeStruct((n, D), jnp.float32),
  )(ev, ob, w1, w2, b.reshape(1, D))


def kernel(object_X, event_X, W_ev, b_ev, g_ev, be_ev, W_ob, b_ob, g_ob, be_ob,
           W1, b1, W2, b2, Wg, bg, node_idx, hedge_idx, main_object, event_sel):
  ev = _proj(event_X, W_ev, b_ev, g_ev, be_ev)
  ob = _proj(object_X, W_ob, b_ob, g_ob, be_ob)
  X = jnp.concatenate([ev, ob], axis=0)

  gn = node_idx.reshape(NW, CH, K)
  gh = hedge_idx.reshape(NW, CH, K)
  ncp, ecp = _sc_counts(gn, gh)

  zn = jnp.zeros((NPAD // NS, D), jnp.float32)
  zh = jnp.zeros((HPAD // NS, D), jnp.float32)

  e1p = _sc_seg(X, gn, gh, zh, opad=HPAD)
  ef1 = _combine(e1p, ecp)
  n1p = _sc_seg(ef1, gh, gn, zn, opad=NPAD)
  H1 = _update(n1p, ncp, X, W1, b1)

  e2p = _sc_seg(H1, gn, gh, zh, opad=HPAD)
  ef2 = _combine(e2p, ecp)
  n2p = _sc_seg(ef2, gh, gn, zn, opad=NPAD)
  H2 = _update(n2p, ncp, H1, W2, b2)

  sel = jnp.concatenate([event_sel, main_object + N_EVENTS], axis=0)
  rows = _sc_gather(H2, sel)
  return _fusion(rows[:1024], rows[1024:], Wg[:D], Wg[D:], bg)
